# Initial kernel scaffold; baseline (speedup 1.0000x reference)
import jax
import jax.numpy as jnp
from jax.experimental import pallas as pl


def kernel(x):
    raise NotImplementedError("write your pallas kernel here")


# speedup vs baseline: 1.0170x; 1.0170x over previous
import jax
import jax.numpy as jnp
from jax.experimental import pallas as pl

_ROWS = 100000
_COLS = 512
_BLOCK_ROWS = 5000


def _copy_assign_block(x_ref, o_ref):
    o_ref[...] = x_ref[...]

    @pl.when(pl.program_id(0) == 0)
    def _():
        o_ref[1:3, :] = jnp.ones((2, _COLS), dtype=o_ref.dtype)


def kernel(x):
    grid = _ROWS // _BLOCK_ROWS
    return pl.pallas_call(
        _copy_assign_block,
        grid=(grid,),
        in_specs=[pl.BlockSpec((_BLOCK_ROWS, _COLS), lambda i: (i, 0))],
        out_specs=pl.BlockSpec((_BLOCK_ROWS, _COLS), lambda i: (i, 0)),
        out_shape=jax.ShapeDtypeStruct((_ROWS, _COLS), x.dtype),
    )(x)
